# Initial kernel scaffold; baseline (speedup 1.0000x reference)
import jax
import jax.numpy as jnp
from jax.experimental import pallas as pl


def kernel(predictions, labels, positions):
    raise NotImplementedError("write your pallas kernel here")


# speedup vs baseline: 297.5873x; 297.5873x over previous
import functools

import jax
import jax.numpy as jnp
from jax import lax
from jax.experimental import pallas as pl
from jax.experimental.pallas import tpu as pltpu
from jax.experimental.pallas import tpu_sc as plsc

B, S, D = 64, 8192, 32
NC, NS, L = 2, 16, 16
NW = NC * NS
BPW = B // NW
CHUNK = 2048
NCHUNK = S // CHUNK
GROUPS = CHUNK // L
KD = 64 * D

_mesh = plsc.VectorSubcoreMesh(core_axis_name="c", subcore_axis_name="s")


@functools.partial(
    pl.kernel,
    out_type=jax.ShapeDtypeStruct((NW, L), jnp.float32),
    mesh=_mesh,
    compiler_params=pltpu.CompilerParams(needs_layout_passes=False),
    scratch_types=[
        pltpu.VMEM((CHUNK * D,), jnp.float32),
        pltpu.VMEM((BPW * KD,), jnp.float32),
        pltpu.VMEM((BPW * S,), jnp.int32),
        pltpu.VMEM((L,), jnp.float32),
    ],
)
def _mse_sc(pred_hbm, lbl_hbm, pos_hbm, out_hbm, pred_v, pos_v, lbl_v, acc_v):
    cid = lax.axis_index("c")
    sid = lax.axis_index("s")
    wid = sid * NC + cid
    acc_v[...] = jnp.zeros((L,), jnp.float32)
    iota = lax.iota(jnp.int32, L)
    iota_d = iota * D
    for bl in range(BPW):
        b = wid * BPW + bl
        pltpu.sync_copy(pos_hbm.at[b], pos_v.at[pl.ds(bl * KD, KD)])
        pltpu.sync_copy(lbl_hbm.at[b], lbl_v.at[pl.ds(bl * S, S)])
    for bl in range(BPW):
        b = wid * BPW + bl
        for c in range(NCHUNK):
            pltpu.sync_copy(pred_hbm.at[b, pl.ds(c * CHUNK * D, CHUNK * D)],
                            pred_v)

            def group_body(g, acc, bl=bl, c=c):
                lbl = lbl_v[pl.ds(bl * S + c * CHUNK + g * L, L)]
                pbase = g * (L * D) + iota_d
                cbase = bl * KD + lbl * D
                for f in range(D):
                    p = plsc.load_gather(pred_v, [pbase + f])
                    ctr = plsc.load_gather(pos_v, [cbase + f])
                    d = p - ctr
                    acc = acc + d * d
                return acc

            acc_v[...] = lax.fori_loop(0, GROUPS, group_body, acc_v[...])
    pltpu.sync_copy(acc_v, out_hbm.at[wid])


def kernel(predictions, labels, positions):
    partials = _mse_sc(
        predictions.reshape(B, S * D),
        labels.astype(jnp.int32),
        positions.reshape(B, KD),
    )
    return jnp.sum(partials) / jnp.float32(B * S)


# speedup vs baseline: 957.4379x; 3.2173x over previous
import functools

import jax
import jax.numpy as jnp
from jax import lax
from jax.experimental import pallas as pl
from jax.experimental.pallas import tpu as pltpu
from jax.experimental.pallas import tpu_sc as plsc

B, S, D = 64, 8192, 32
NC, NS, L = 2, 16, 16
NW = NC * NS
BPW = B // NW
CHUNK = 1024
NCHUNK = S // CHUNK
TOT = BPW * NCHUNK
GROUPS = CHUNK // L
KD = 64 * D

_mesh = plsc.VectorSubcoreMesh(core_axis_name="c", subcore_axis_name="s")


@functools.partial(
    pl.kernel,
    out_type=jax.ShapeDtypeStruct((NW, L), jnp.float32),
    mesh=_mesh,
    compiler_params=pltpu.CompilerParams(needs_layout_passes=False),
    scratch_types=[
        pltpu.VMEM((CHUNK * D,), jnp.float32),
        pltpu.VMEM((CHUNK * D,), jnp.float32),
        pltpu.VMEM((BPW * KD,), jnp.float32),
        pltpu.VMEM((BPW * S,), jnp.int32),
        pltpu.VMEM((L,), jnp.float32),
        pltpu.SemaphoreType.DMA,
        pltpu.SemaphoreType.DMA,
    ],
)
def _mse_sc(pred_hbm, lbl_hbm, pos_hbm, out_hbm,
            pred_v0, pred_v1, pos_v, lbl_v, acc_v, sem0, sem1):
    cid = lax.axis_index("c")
    sid = lax.axis_index("s")
    wid = sid * NC + cid
    acc_v[...] = jnp.zeros((L,), jnp.float32)
    iota = lax.iota(jnp.int32, L)
    for bl in range(BPW):
        b = wid * BPW + bl
        pltpu.sync_copy(pos_hbm.at[b], pos_v.at[pl.ds(bl * KD, KD)])
        pltpu.sync_copy(lbl_hbm.at[b], lbl_v.at[pl.ds(bl * S, S)])

    bufs = (pred_v0, pred_v1)
    sems = (sem0, sem1)

    def chunk_src(k):
        bl, c = divmod(k, NCHUNK)
        b = wid * BPW + bl
        return pred_hbm.at[b, pl.ds(c * CHUNK * D, CHUNK * D)]

    copies = [pltpu.async_copy(chunk_src(0), bufs[0], sems[0]), None]
    for k in range(TOT):
        j = k & 1
        nj = (k + 1) & 1
        if k + 1 < TOT:
            copies[nj] = pltpu.async_copy(chunk_src(k + 1), bufs[nj], sems[nj])
        copies[j].wait()
        bl, c = divmod(k, NCHUNK)
        buf = bufs[j]

        def group_body(g, acc, bl=bl, c=c, buf=buf):
            lbl_vec = lbl_v[pl.ds(bl * S + c * CHUNK + g * L, L)]
            lbl_base = lbl_vec * D + (bl * KD)
            for t in range(L):
                bvec = jnp.take_along_axis(
                    lbl_base, jnp.full((L,), t, jnp.int32), axis=0)
                cidx = bvec + iota
                tok = (g * L + t) * D
                p0 = buf[pl.ds(tok, L)]
                p1 = buf[pl.ds(tok + L, L)]
                c0 = plsc.load_gather(pos_v, [cidx])
                c1 = plsc.load_gather(pos_v, [cidx + L])
                d0 = p0 - c0
                d1 = p1 - c1
                acc = acc + d0 * d0 + d1 * d1
            return acc

        acc_v[...] = lax.fori_loop(0, GROUPS, group_body, acc_v[...])
    pltpu.sync_copy(acc_v, out_hbm.at[wid])


def kernel(predictions, labels, positions):
    partials = _mse_sc(
        predictions.reshape(B, S * D),
        labels.astype(jnp.int32),
        positions.reshape(B, KD),
    )
    return jnp.sum(partials) / jnp.float32(B * S)
